# baseline (speedup 1.0000x reference)
import numpy as np
import jax
import jax.numpy as jnp
from jax.experimental import pallas as pl
from jax.experimental.pallas import tpu as pltpu

N = 2048
NW = 4096
_HI = jax.lax.Precision.HIGHEST

_k = np.arange(N)
_COS_NP = np.cos((2.0 * np.pi / N) * (np.outer(_k, _k) % N)).astype(np.float32)
_T2048_NP = np.asarray([max(1, int(c * (1.0 - 0.3))) for c in range(N + 1)],
                       dtype=np.float32).reshape(1, N + 1)
_T4096_NP = np.asarray([max(1, int(c * (1.0 - 0.3))) for c in range(NW + 1)],
                       dtype=np.float32).reshape(1, NW + 1)


def _sumsq_kernel(w_ref, te_ref):
    a = w_ref[...]
    te_ref[...] = jnp.sum(a * a, axis=1, keepdims=True)


def _time_energy(w):
    bi = 512
    return pl.pallas_call(
        _sumsq_kernel,
        grid=(NW // bi,),
        in_specs=[pl.BlockSpec((bi, N), lambda i: (i, 0))],
        out_specs=pl.BlockSpec((bi, 1), lambda i: (i, 0)),
        out_shape=jax.ShapeDtypeStruct((NW, 1), jnp.float32),
    )(w)


def _gram_kernel(a_ref, b_ref, c_ref):
    k = pl.program_id(2)

    @pl.when(k == 0)
    def _():
        c_ref[...] = jnp.zeros_like(c_ref)

    c_ref[...] += jax.lax.dot_general(
        a_ref[...], b_ref[...], (((0,), (0,)), ((), ())),
        preferred_element_type=jnp.float32, precision=_HI)


def _gram(w):
    bi = bj = 512
    bk = 1024
    return pl.pallas_call(
        _gram_kernel,
        grid=(N // bi, N // bj, NW // bk),
        in_specs=[
            pl.BlockSpec((bk, bi), lambda i, j, k: (k, i)),
            pl.BlockSpec((bk, bj), lambda i, j, k: (k, j)),
        ],
        out_specs=pl.BlockSpec((bi, bj), lambda i, j, k: (i, j)),
        out_shape=jax.ShapeDtypeStruct((N, N), jnp.float32),
    )(w, w)


_BSKEW = 256


def _lroll(x, s):
    return jnp.concatenate([x[:, s:], x[:, :s]], axis=1)


def _skew_kernel(c_ref, r_ref):
    blk = pl.program_id(0)

    @pl.when(blk == 0)
    def _():
        r_ref[...] = jnp.zeros_like(r_ref)

    x = c_ref[...]
    rows = _BSKEW
    while rows > 1:
        half = rows // 2
        top = jax.lax.slice(x, (0, 0), (half, N))
        bot = jax.lax.slice(x, (half, 0), (rows, N))
        x = top + _lroll(bot, half)
        rows = half
    for t in (8, 9, 10):
        s = 1 << t
        bit = (blk >> (t - 8)) & 1
        x = jnp.where(bit == 1, _lroll(x, s), x)
    r_ref[...] += x


def _diag_sums(c):
    return pl.pallas_call(
        _skew_kernel,
        grid=(N // _BSKEW,),
        in_specs=[pl.BlockSpec((_BSKEW, N), lambda i: (i, 0))],
        out_specs=pl.BlockSpec((1, N), lambda i: (0, 0)),
        out_shape=jax.ShapeDtypeStruct((1, N), jnp.float32),
    )(c)


def _matvec_kernel(r_ref, cos_ref, g_ref):
    g_ref[...] = jnp.dot(r_ref[...], cos_ref[...],
                         preferred_element_type=jnp.float32, precision=_HI)


def _freq_energy(r, cos):
    return pl.pallas_call(
        _matvec_kernel,
        in_specs=[pl.BlockSpec((1, N), lambda: (0, 0)),
                  pl.BlockSpec((N, N), lambda: (0, 0))],
        out_specs=pl.BlockSpec((1, N), lambda: (0, 0)),
        out_shape=jax.ShapeDtypeStruct((1, N), jnp.float32),
    )(r, cos)


def _topk_mask_cols(v_row, v_col, table, L, chunk):
    mx = jnp.max(v_row)
    thresh_col = (v_col > 0.01 * mx).astype(jnp.float32)
    cnt = jnp.sum(thresh_col)
    ti = jax.lax.broadcasted_iota(jnp.int32, (1, L + 1), 1)
    target = jnp.sum(jnp.where(ti == cnt.astype(jnp.int32), table, 0.0))
    ranks = []
    for r0 in range(0, L, chunk):
        vc = jax.lax.slice(v_col, (r0, 0), (r0 + chunk, 1))
        gt = (v_row > vc).astype(jnp.float32)
        ci = jax.lax.broadcasted_iota(jnp.int32, (chunk, L), 1)
        ri = jax.lax.broadcasted_iota(jnp.int32, (chunk, L), 0) + r0
        tie = jnp.where((v_row == vc) & (ci < ri), 1.0, 0.0)
        ranks.append(jnp.sum(gt + tie, axis=1, keepdims=True))
    rank = jnp.concatenate(ranks, axis=0)
    mask_top = (rank < target).astype(jnp.float32)
    return jnp.where(target < cnt, mask_top, thresh_col)


def _mask_kernel(g_row_ref, g_col_ref, te_row_ref, te_col_ref, cos_ref,
                 t2048_ref, t4096_ref, c_ref, tm_ref):
    fm_col = _topk_mask_cols(g_row_ref[...], g_col_ref[...], t2048_ref[...],
                             N, 1024)
    tm_ref[...] = _topk_mask_cols(te_row_ref[...], te_col_ref[...],
                                  t4096_ref[...], NW, 512)
    c_ref[...] = jax.lax.dot_general(
        fm_col, cos_ref[...], (((0,), (0,)), ((), ())),
        preferred_element_type=jnp.float32, precision=_HI) * jnp.float32(1.0 / N)


def _masks(g_row, g_col, te_row, te_col, cos, t2048, t4096):
    return pl.pallas_call(
        _mask_kernel,
        in_specs=[
            pl.BlockSpec((1, N), lambda: (0, 0)),
            pl.BlockSpec((N, 1), lambda: (0, 0)),
            pl.BlockSpec((1, NW), lambda: (0, 0)),
            pl.BlockSpec((NW, 1), lambda: (0, 0)),
            pl.BlockSpec((N, N), lambda: (0, 0)),
            pl.BlockSpec((1, N + 1), lambda: (0, 0)),
            pl.BlockSpec((1, NW + 1), lambda: (0, 0)),
        ],
        out_specs=[pl.BlockSpec((1, N), lambda: (0, 0)),
                   pl.BlockSpec((NW, 1), lambda: (0, 0))],
        out_shape=[jax.ShapeDtypeStruct((1, N), jnp.float32),
                   jax.ShapeDtypeStruct((NW, 1), jnp.float32)],
    )(g_row, g_col, te_row, te_col, cos, t2048, t4096)


def _rroll(x, s):
    return jnp.concatenate([x[:, N - s:], x[:, :N - s]], axis=1)


def _circ_kernel(c_ref, m_ref):
    blk = pl.program_id(0)
    base = c_ref[...]
    for t in (8, 9, 10):
        s = 1 << t
        bit = (blk >> (t - 8)) & 1
        base = jnp.where(bit == 1, _rroll(base, s), base)
    m = base
    rows = 1
    while rows < _BSKEW:
        m = jnp.concatenate([m, _rroll(m, rows)], axis=0)
        rows *= 2
    m_ref[...] = m


def _circulant(c):
    return pl.pallas_call(
        _circ_kernel,
        grid=(N // _BSKEW,),
        in_specs=[pl.BlockSpec((1, N), lambda i: (0, 0))],
        out_specs=pl.BlockSpec((_BSKEW, N), lambda i: (i, 0)),
        out_shape=jax.ShapeDtypeStruct((N, N), jnp.float32),
    )(c)


def _final_kernel(w_ref, m_ref, tm_ref, o_ref):
    k = pl.program_id(2)

    @pl.when(k == 0)
    def _():
        o_ref[...] = jnp.zeros_like(o_ref)

    a = w_ref[...] * tm_ref[...]
    o_ref[...] += jnp.dot(a, m_ref[...],
                          preferred_element_type=jnp.float32,
                          precision=jax.lax.Precision.DEFAULT)


def _reconstruct(w, m, tm):
    bi = 512
    bj = 1024
    bk = 1024
    return pl.pallas_call(
        _final_kernel,
        grid=(NW // bi, N // bj, N // bk),
        in_specs=[
            pl.BlockSpec((bi, bk), lambda i, j, k: (i, k)),
            pl.BlockSpec((bk, bj), lambda i, j, k: (k, j)),
            pl.BlockSpec((bi, 1), lambda i, j, k: (i, 0)),
        ],
        out_specs=pl.BlockSpec((bi, bj), lambda i, j, k: (i, j)),
        out_shape=jax.ShapeDtypeStruct((NW, N), jnp.float32),
    )(w, m, tm)


def kernel(weight):
    w = weight.reshape(NW, N).astype(jnp.float32)
    cos = jnp.asarray(_COS_NP)
    te_col = _time_energy(w)
    c_gram = _gram(w)
    r = _diag_sums(c_gram)
    g_row = _freq_energy(r, cos)
    g_col = g_row.reshape(N, 1)
    te_row = te_col.reshape(1, NW)
    c_filt, tm = _masks(g_row, g_col, te_row, te_col, cos,
                        jnp.asarray(_T2048_NP), jnp.asarray(_T4096_NP))
    m = _circulant(c_filt)
    rec = _reconstruct(w, m, tm)
    return rec.reshape(weight.shape)


# speedup vs baseline: 1.0789x; 1.0789x over previous
import numpy as np
import jax
import jax.numpy as jnp
from jax.experimental import pallas as pl
from jax.experimental.pallas import tpu as pltpu

N = 2048
NW = 4096
_HI = jax.lax.Precision.HIGHEST

_k = np.arange(N)
_COS_NP = np.cos((2.0 * np.pi / N) * (np.outer(_k, _k) % N)).astype(np.float32)
_T2048_NP = np.asarray([max(1, int(c * (1.0 - 0.3))) for c in range(N + 1)],
                       dtype=np.float32).reshape(1, N + 1)
_T4096_NP = np.asarray([max(1, int(c * (1.0 - 0.3))) for c in range(NW + 1)],
                       dtype=np.float32).reshape(1, NW + 1)


def _sumsq_kernel(w_ref, te_ref):
    a = w_ref[...]
    te_ref[...] = jnp.sum(a * a, axis=1, keepdims=True)


def _time_energy(w):
    bi = 512
    return pl.pallas_call(
        _sumsq_kernel,
        grid=(NW // bi,),
        in_specs=[pl.BlockSpec((bi, N), lambda i: (i, 0))],
        out_specs=pl.BlockSpec((bi, 1), lambda i: (i, 0)),
        out_shape=jax.ShapeDtypeStruct((NW, 1), jnp.float32),
    )(w)


def _gram_kernel(a_ref, b_ref, c_ref):
    k = pl.program_id(2)

    @pl.when(k == 0)
    def _():
        c_ref[...] = jnp.zeros_like(c_ref)

    c_ref[...] += jax.lax.dot_general(
        a_ref[...], b_ref[...], (((0,), (0,)), ((), ())),
        preferred_element_type=jnp.float32, precision=_HI)


def _gram(w):
    bi = bj = 2048
    bk = 512
    return pl.pallas_call(
        _gram_kernel,
        grid=(N // bi, N // bj, NW // bk),
        in_specs=[
            pl.BlockSpec((bk, bi), lambda i, j, k: (k, i)),
            pl.BlockSpec((bk, bj), lambda i, j, k: (k, j)),
        ],
        out_specs=pl.BlockSpec((bi, bj), lambda i, j, k: (i, j)),
        out_shape=jax.ShapeDtypeStruct((N, N), jnp.float32),
    )(w, w)


_BSKEW = 256


def _lroll(x, s):
    return jnp.concatenate([x[:, s:], x[:, :s]], axis=1)


def _skew_kernel(c_ref, r_ref):
    blk = pl.program_id(0)

    @pl.when(blk == 0)
    def _():
        r_ref[...] = jnp.zeros_like(r_ref)

    x = c_ref[...]
    rows = _BSKEW
    while rows > 1:
        half = rows // 2
        top = jax.lax.slice(x, (0, 0), (half, N))
        bot = jax.lax.slice(x, (half, 0), (rows, N))
        x = top + _lroll(bot, half)
        rows = half
    for t in (8, 9, 10):
        s = 1 << t
        bit = (blk >> (t - 8)) & 1
        x = jnp.where(bit == 1, _lroll(x, s), x)
    r_ref[...] += x


def _diag_sums(c):
    return pl.pallas_call(
        _skew_kernel,
        grid=(N // _BSKEW,),
        in_specs=[pl.BlockSpec((_BSKEW, N), lambda i: (i, 0))],
        out_specs=pl.BlockSpec((1, N), lambda i: (0, 0)),
        out_shape=jax.ShapeDtypeStruct((1, N), jnp.float32),
    )(c)


def _matvec_kernel(r_ref, cos_ref, g_ref):
    g_ref[...] = jnp.dot(r_ref[...], cos_ref[...],
                         preferred_element_type=jnp.float32, precision=_HI)


def _freq_energy(r, cos):
    return pl.pallas_call(
        _matvec_kernel,
        in_specs=[pl.BlockSpec((1, N), lambda: (0, 0)),
                  pl.BlockSpec((N, N), lambda: (0, 0))],
        out_specs=pl.BlockSpec((1, N), lambda: (0, 0)),
        out_shape=jax.ShapeDtypeStruct((1, N), jnp.float32),
    )(r, cos)


def _topk_mask_cols(v_row, v_col, table, L, chunk):
    mx = jnp.max(v_row)
    thresh_col = (v_col > 0.01 * mx).astype(jnp.float32)
    cnt = jnp.sum(thresh_col)
    ti = jax.lax.broadcasted_iota(jnp.int32, (1, L + 1), 1)
    target = jnp.sum(jnp.where(ti == cnt.astype(jnp.int32), table, 0.0))
    ranks = []
    for r0 in range(0, L, chunk):
        vc = jax.lax.slice(v_col, (r0, 0), (r0 + chunk, 1))
        gt = (v_row > vc).astype(jnp.float32)
        ci = jax.lax.broadcasted_iota(jnp.int32, (chunk, L), 1)
        ri = jax.lax.broadcasted_iota(jnp.int32, (chunk, L), 0) + r0
        tie = jnp.where((v_row == vc) & (ci < ri), 1.0, 0.0)
        ranks.append(jnp.sum(gt + tie, axis=1, keepdims=True))
    rank = jnp.concatenate(ranks, axis=0)
    mask_top = (rank < target).astype(jnp.float32)
    return jnp.where(target < cnt, mask_top, thresh_col)


def _mask_kernel(g_row_ref, g_col_ref, te_row_ref, te_col_ref, cos_ref,
                 t2048_ref, t4096_ref, c_ref, tm_ref):
    fm_col = _topk_mask_cols(g_row_ref[...], g_col_ref[...], t2048_ref[...],
                             N, 1024)
    tm_ref[...] = _topk_mask_cols(te_row_ref[...], te_col_ref[...],
                                  t4096_ref[...], NW, 512)
    c_ref[...] = jax.lax.dot_general(
        fm_col, cos_ref[...], (((0,), (0,)), ((), ())),
        preferred_element_type=jnp.float32, precision=_HI) * jnp.float32(1.0 / N)


def _masks(g_row, g_col, te_row, te_col, cos, t2048, t4096):
    return pl.pallas_call(
        _mask_kernel,
        in_specs=[
            pl.BlockSpec((1, N), lambda: (0, 0)),
            pl.BlockSpec((N, 1), lambda: (0, 0)),
            pl.BlockSpec((1, NW), lambda: (0, 0)),
            pl.BlockSpec((NW, 1), lambda: (0, 0)),
            pl.BlockSpec((N, N), lambda: (0, 0)),
            pl.BlockSpec((1, N + 1), lambda: (0, 0)),
            pl.BlockSpec((1, NW + 1), lambda: (0, 0)),
        ],
        out_specs=[pl.BlockSpec((1, N), lambda: (0, 0)),
                   pl.BlockSpec((NW, 1), lambda: (0, 0))],
        out_shape=[jax.ShapeDtypeStruct((1, N), jnp.float32),
                   jax.ShapeDtypeStruct((NW, 1), jnp.float32)],
    )(g_row, g_col, te_row, te_col, cos, t2048, t4096)


def _rroll(x, s):
    return jnp.concatenate([x[:, N - s:], x[:, :N - s]], axis=1)


def _circ_kernel(c_ref, m_ref):
    blk = pl.program_id(0)
    base = c_ref[...]
    for t in (8, 9, 10):
        s = 1 << t
        bit = (blk >> (t - 8)) & 1
        base = jnp.where(bit == 1, _rroll(base, s), base)
    m = base
    rows = 1
    while rows < _BSKEW:
        m = jnp.concatenate([m, _rroll(m, rows)], axis=0)
        rows *= 2
    m_ref[...] = m


def _circulant(c):
    return pl.pallas_call(
        _circ_kernel,
        grid=(N // _BSKEW,),
        in_specs=[pl.BlockSpec((1, N), lambda i: (0, 0))],
        out_specs=pl.BlockSpec((_BSKEW, N), lambda i: (i, 0)),
        out_shape=jax.ShapeDtypeStruct((N, N), jnp.float32),
    )(c)


def _final_kernel(w_ref, m_ref, tm_ref, o_ref):
    k = pl.program_id(2)

    @pl.when(k == 0)
    def _():
        o_ref[...] = jnp.zeros_like(o_ref)

    a = w_ref[...] * tm_ref[...]
    o_ref[...] += jnp.dot(a, m_ref[...],
                          preferred_element_type=jnp.float32,
                          precision=jax.lax.Precision.DEFAULT)


def _reconstruct(w, m, tm):
    bi = 2048
    bj = 2048
    bk = 512
    return pl.pallas_call(
        _final_kernel,
        grid=(NW // bi, N // bj, N // bk),
        in_specs=[
            pl.BlockSpec((bi, bk), lambda i, j, k: (i, k)),
            pl.BlockSpec((bk, bj), lambda i, j, k: (k, j)),
            pl.BlockSpec((bi, 1), lambda i, j, k: (i, 0)),
        ],
        out_specs=pl.BlockSpec((bi, bj), lambda i, j, k: (i, j)),
        out_shape=jax.ShapeDtypeStruct((NW, N), jnp.float32),
    )(w, m, tm)


def kernel(weight):
    w = weight.reshape(NW, N).astype(jnp.float32)
    cos = jnp.asarray(_COS_NP)
    te_col = _time_energy(w)
    c_gram = _gram(w)
    r = _diag_sums(c_gram)
    g_row = _freq_energy(r, cos)
    g_col = g_row.reshape(N, 1)
    te_row = te_col.reshape(1, NW)
    c_filt, tm = _masks(g_row, g_col, te_row, te_col, cos,
                        jnp.asarray(_T2048_NP), jnp.asarray(_T4096_NP))
    m = _circulant(c_filt)
    rec = _reconstruct(w, m, tm)
    return rec.reshape(weight.shape)


# speedup vs baseline: 1.1220x; 1.0400x over previous
import numpy as np
import jax
import jax.numpy as jnp
from jax.experimental import pallas as pl
from jax.experimental.pallas import tpu as pltpu

N = 2048
NW = 4096
_HI = jax.lax.Precision.HIGHEST

_k = np.arange(N)
_COS_NP = np.cos((2.0 * np.pi / N) * (np.outer(_k, _k) % N)).astype(np.float32)
_T2048_NP = np.asarray([max(1, int(c * (1.0 - 0.3))) for c in range(N + 1)],
                       dtype=np.float32).reshape(1, N + 1)
_T4096_NP = np.asarray([max(1, int(c * (1.0 - 0.3))) for c in range(NW + 1)],
                       dtype=np.float32).reshape(1, NW + 1)


def _gram_kernel(a_ref, c_ref, te_ref):
    k = pl.program_id(0)

    @pl.when(k == 0)
    def _():
        c_ref[...] = jnp.zeros_like(c_ref)

    a = a_ref[...]
    c_ref[...] += jax.lax.dot_general(
        a, a, (((0,), (0,)), ((), ())),
        preferred_element_type=jnp.float32, precision=_HI)
    te_ref[...] = jnp.sum(a * a, axis=1, keepdims=True)


def _gram(w):
    bk = 512
    return pl.pallas_call(
        _gram_kernel,
        grid=(NW // bk,),
        in_specs=[pl.BlockSpec((bk, N), lambda k: (k, 0))],
        out_specs=[pl.BlockSpec((N, N), lambda k: (0, 0)),
                   pl.BlockSpec((bk, 1), lambda k: (k, 0))],
        out_shape=[jax.ShapeDtypeStruct((N, N), jnp.float32),
                   jax.ShapeDtypeStruct((NW, 1), jnp.float32)],
    )(w)


def _lroll(x, s):
    return jnp.concatenate([x[:, s:], x[:, :s]], axis=1)


def _rroll(x, s):
    return jnp.concatenate([x[:, N - s:], x[:, :N - s]], axis=1)


def _fold_block(x, blk, rows):
    while rows > 1:
        half = rows // 2
        top = jax.lax.slice(x, (0, 0), (half, N))
        bot = jax.lax.slice(x, (half, 0), (rows, N))
        x = top + _lroll(bot, half)
        rows = half
    for t in (8, 9, 10):
        bit = (blk >> (t - 8)) & 1
        x = jnp.where(bit == 1, _lroll(x, 1 << t), x)
    return x


def _ordered_key(v):
    i = jax.lax.bitcast_convert_type(v, jnp.int32)
    return jnp.where(i >= 0, i, i ^ jnp.int32(0x7FFFFFFF))


def _prefix_excl(eq, s0, s1):
    p = eq
    s = 1
    while s < s1:
        shifted = jnp.concatenate(
            [jnp.zeros((s0, s), jnp.float32), jax.lax.slice(p, (0, 0), (s0, s1 - s))],
            axis=1)
        p = p + shifted
        s *= 2
    rs = jnp.sum(eq, axis=1, keepdims=True)
    rp = rs
    s = 1
    while s < s0:
        shifted = jnp.concatenate(
            [jnp.zeros((s, 1), jnp.float32), jax.lax.slice(rp, (0, 0), (s0 - s, 1))],
            axis=0)
        rp = rp + shifted
        s *= 2
    return (p - eq) + (rp - rs)


def _topk_mask(v, table, L, s0, s1):
    mx = jnp.max(v)
    thresh = (v > jnp.float32(0.01) * mx)
    cnt = jnp.sum(thresh.astype(jnp.int32))
    ti = jax.lax.broadcasted_iota(jnp.int32, (1, L + 1), 1)
    target = jnp.sum(jnp.where(ti == cnt, table, 0.0)).astype(jnp.int32)
    key = _ordered_key(v)

    def body(_, lohi):
        lo, hi = lohi
        mid = lo + (hi - lo) // 2
        cgt = jnp.sum((key >= mid).astype(jnp.int32))
        pred = cgt >= target
        return (jnp.where(pred, mid, lo), jnp.where(pred, hi, mid))

    lo0 = jnp.min(key)
    hi0 = jnp.max(key) + 1
    tau, _ = jax.lax.fori_loop(0, 32, body, (lo0, hi0))
    gt = key > tau
    eq = (key == tau).astype(jnp.float32)
    fill = (target - jnp.sum(gt.astype(jnp.int32))).astype(jnp.float32)
    prefix = _prefix_excl(eq, s0, s1)
    mask_top = (gt | ((eq > 0) & (prefix < fill))).astype(jnp.float32)
    return jnp.where(target < cnt, mask_top, thresh.astype(jnp.float32))


def _matvec_cos(vec, cos_ref, blk=512):
    parts = []
    for j0 in range(0, N, blk):
        acc = jnp.zeros((1, blk), jnp.float32)
        for k0 in range(0, N, blk):
            vs = jax.lax.slice(vec, (0, k0), (1, k0 + blk))
            acc += jnp.dot(vs, cos_ref[k0:k0 + blk, j0:j0 + blk],
                           preferred_element_type=jnp.float32, precision=_HI)
        parts.append(acc)
    return jnp.concatenate(parts, axis=1)


def _small_kernel(c_ref, cos_ref, te8_ref, t2048_ref, t4096_ref,
                  cv_ref, tm_ref, r_scr):
    i = pl.program_id(0)

    @pl.when(i == 0)
    def _():
        r_scr[...] = jnp.zeros_like(r_scr)

    @pl.when(i <= 7)
    def _():
        r_scr[...] += _fold_block(c_ref[...], i, 256)

    @pl.when(i == 8)
    def _():
        g = _matvec_cos(r_scr[...], cos_ref)
        fm = _topk_mask(g, t2048_ref[...], N, 1, N)
        cv_ref[...] = _matvec_cos(fm, cos_ref) * jnp.float32(1.0 / N)
        tm_ref[...] = _topk_mask(te8_ref[...], t4096_ref[...], NW, 8, 512)


def _small_stages(c, cos, te8, t2048, t4096):
    return pl.pallas_call(
        _small_kernel,
        grid=(9,),
        in_specs=[
            pl.BlockSpec((256, N), lambda i: (jnp.minimum(i, 7), 0)),
            pl.BlockSpec((N, N), lambda i: (0, 0)),
            pl.BlockSpec((8, 512), lambda i: (0, 0)),
            pl.BlockSpec((1, N + 1), lambda i: (0, 0)),
            pl.BlockSpec((1, NW + 1), lambda i: (0, 0)),
        ],
        out_specs=[pl.BlockSpec((1, N), lambda i: (0, 0)),
                   pl.BlockSpec((8, 512), lambda i: (0, 0))],
        out_shape=[jax.ShapeDtypeStruct((1, N), jnp.float32),
                   jax.ShapeDtypeStruct((8, 512), jnp.float32)],
        scratch_shapes=[pltpu.VMEM((1, N), jnp.float32)],
    )(c, cos, te8, t2048, t4096)


def _final_kernel(w_ref, cv_ref, tm_ref, o_ref, m_scr):
    i = pl.program_id(0)
    k = pl.program_id(1)

    @pl.when(k == 0)
    def _():
        o_ref[...] = jnp.zeros_like(o_ref)

    @pl.when(i == 0)
    def _():
        base = cv_ref[...]
        base = jnp.where(k & 1 == 1, _rroll(base, 512), base)
        base = jnp.where(k & 2 == 2, _rroll(base, 1024), base)
        m = base
        rows = 1
        while rows < 512:
            m = jnp.concatenate([m, _rroll(m, rows)], axis=0)
            rows *= 2
        m_scr[pl.ds(k * 512, 512), :] = m

    a = w_ref[...] * tm_ref[...]
    o_ref[...] += jnp.dot(a, m_scr[pl.ds(k * 512, 512), :],
                          preferred_element_type=jnp.float32,
                          precision=jax.lax.Precision.DEFAULT)


def _reconstruct(w, cvec, tm):
    bi = 1024
    bk = 512
    return pl.pallas_call(
        _final_kernel,
        grid=(NW // bi, N // bk),
        in_specs=[
            pl.BlockSpec((bi, bk), lambda i, k: (i, k)),
            pl.BlockSpec((1, N), lambda i, k: (0, 0)),
            pl.BlockSpec((bi, 1), lambda i, k: (i, 0)),
        ],
        out_specs=pl.BlockSpec((bi, N), lambda i, k: (i, 0)),
        out_shape=jax.ShapeDtypeStruct((NW, N), jnp.float32),
        scratch_shapes=[pltpu.VMEM((N, N), jnp.float32)],
    )(w, cvec, tm)


def kernel(weight):
    w = weight.reshape(NW, N).astype(jnp.float32)
    cos = jnp.asarray(_COS_NP)
    c_gram, te_col = _gram(w)
    cvec, tm8 = _small_stages(c_gram, cos, te_col.reshape(8, 512),
                              jnp.asarray(_T2048_NP), jnp.asarray(_T4096_NP))
    rec = _reconstruct(w, cvec, tm8.reshape(NW, 1))
    return rec.reshape(weight.shape)


# speedup vs baseline: 1.6419x; 1.4633x over previous
import numpy as np
import jax
import jax.numpy as jnp
from jax.experimental import pallas as pl
from jax.experimental.pallas import tpu as pltpu

N = 2048
NW = 4096
_HI = jax.lax.Precision.HIGHEST

_k = np.arange(N)
_COS_NP = np.cos((2.0 * np.pi / N) * (np.outer(_k, _k) % N)).astype(np.float32)
_T2048_NP = np.asarray([max(1, int(c * (1.0 - 0.3))) for c in range(N + 1)],
                       dtype=np.float32).reshape(1, N + 1)
_T4096_NP = np.asarray([max(1, int(c * (1.0 - 0.3))) for c in range(NW + 1)],
                       dtype=np.float32).reshape(1, NW + 1)


def _gram_kernel(a_ref, c_ref, te_ref):
    k = pl.program_id(0)

    @pl.when(k == 0)
    def _():
        c_ref[...] = jnp.zeros_like(c_ref)

    a = a_ref[...]
    hi = a.astype(jnp.bfloat16).astype(jnp.float32)
    lo = a - hi
    dims = (((0,), (0,)), ((), ()))
    dflt = jax.lax.Precision.DEFAULT
    acc = jax.lax.dot_general(hi, hi, dims, preferred_element_type=jnp.float32,
                              precision=dflt)
    acc += jax.lax.dot_general(hi, lo, dims, preferred_element_type=jnp.float32,
                               precision=dflt)
    acc += jax.lax.dot_general(lo, hi, dims, preferred_element_type=jnp.float32,
                               precision=dflt)
    c_ref[...] += acc
    te_ref[...] = jnp.sum(a * a, axis=1, keepdims=True)


def _gram(w):
    bk = 512
    return pl.pallas_call(
        _gram_kernel,
        grid=(NW // bk,),
        in_specs=[pl.BlockSpec((bk, N), lambda k: (k, 0))],
        out_specs=[pl.BlockSpec((N, N), lambda k: (0, 0)),
                   pl.BlockSpec((bk, 1), lambda k: (k, 0))],
        out_shape=[jax.ShapeDtypeStruct((N, N), jnp.float32),
                   jax.ShapeDtypeStruct((NW, 1), jnp.float32)],
    )(w)


def _lroll(x, s):
    return jnp.concatenate([x[:, s:], x[:, :s]], axis=1)


def _rroll(x, s):
    return jnp.concatenate([x[:, N - s:], x[:, :N - s]], axis=1)


def _fold_block(x, blk, rows):
    while rows > 1:
        half = rows // 2
        top = jax.lax.slice(x, (0, 0), (half, N))
        bot = jax.lax.slice(x, (half, 0), (rows, N))
        x = top + _lroll(bot, half)
        rows = half
    for t in (8, 9, 10):
        bit = (blk >> (t - 8)) & 1
        x = jnp.where(bit == 1, _lroll(x, 1 << t), x)
    return x


def _ordered_key(v):
    i = jax.lax.bitcast_convert_type(v, jnp.int32)
    return jnp.where(i >= 0, i, i ^ jnp.int32(0x7FFFFFFF))


def _prefix_excl(eq, s0, s1):
    p = eq
    s = 1
    while s < s1:
        shifted = jnp.concatenate(
            [jnp.zeros((s0, s), jnp.float32), jax.lax.slice(p, (0, 0), (s0, s1 - s))],
            axis=1)
        p = p + shifted
        s *= 2
    rs = jnp.sum(eq, axis=1, keepdims=True)
    rp = rs
    s = 1
    while s < s0:
        shifted = jnp.concatenate(
            [jnp.zeros((s, 1), jnp.float32), jax.lax.slice(rp, (0, 0), (s0 - s, 1))],
            axis=0)
        rp = rp + shifted
        s *= 2
    return (p - eq) + (rp - rs)


def _topk_mask(v, table, L, s0, s1):
    mx = jnp.max(v)
    thresh = (v > jnp.float32(0.01) * mx)
    cnt = jnp.sum(thresh.astype(jnp.int32))
    ti = jax.lax.broadcasted_iota(jnp.int32, (1, L + 1), 1)
    target = jnp.sum(jnp.where(ti == cnt, table, 0.0)).astype(jnp.int32)
    key = _ordered_key(v)

    def body(_, lohi):
        lo, hi = lohi
        mid = lo + (hi - lo) // 2
        cgt = jnp.sum((key >= mid).astype(jnp.int32))
        pred = cgt >= target
        return (jnp.where(pred, mid, lo), jnp.where(pred, hi, mid))

    lo0 = jnp.min(key)
    hi0 = jnp.max(key) + 1
    tau, _ = jax.lax.fori_loop(0, 32, body, (lo0, hi0))
    gt = key > tau
    eq = (key == tau).astype(jnp.float32)
    fill = (target - jnp.sum(gt.astype(jnp.int32))).astype(jnp.float32)
    prefix = _prefix_excl(eq, s0, s1)
    mask_top = (gt | ((eq > 0) & (prefix < fill))).astype(jnp.float32)
    return jnp.where(target < cnt, mask_top, thresh.astype(jnp.float32))


def _matvec_cos(vec, cos_ref, blk=512):
    parts = []
    for j0 in range(0, N, blk):
        acc = jnp.zeros((1, blk), jnp.float32)
        for k0 in range(0, N, blk):
            vs = jax.lax.slice(vec, (0, k0), (1, k0 + blk))
            acc += jnp.dot(vs, cos_ref[k0:k0 + blk, j0:j0 + blk],
                           preferred_element_type=jnp.float32, precision=_HI)
        parts.append(acc)
    return jnp.concatenate(parts, axis=1)


def _small_kernel(c_ref, cos_ref, te8_ref, t2048_ref, t4096_ref,
                  cv_ref, tm_ref, r_scr):
    i = pl.program_id(0)

    @pl.when(i == 0)
    def _():
        r_scr[...] = jnp.zeros_like(r_scr)

    @pl.when(i <= 7)
    def _():
        r_scr[...] += _fold_block(c_ref[...], i, 256)

    @pl.when(i == 8)
    def _():
        g = _matvec_cos(r_scr[...], cos_ref)
        fm = _topk_mask(g, t2048_ref[...], N, 1, N)
        cv_ref[...] = _matvec_cos(fm, cos_ref) * jnp.float32(1.0 / N)
        tm_ref[...] = _topk_mask(te8_ref[...], t4096_ref[...], NW, 8, 512)


def _small_stages(c, cos, te8, t2048, t4096):
    return pl.pallas_call(
        _small_kernel,
        grid=(9,),
        in_specs=[
            pl.BlockSpec((256, N), lambda i: (jnp.minimum(i, 7), 0)),
            pl.BlockSpec((N, N), lambda i: (0, 0)),
            pl.BlockSpec((8, 512), lambda i: (0, 0)),
            pl.BlockSpec((1, N + 1), lambda i: (0, 0)),
            pl.BlockSpec((1, NW + 1), lambda i: (0, 0)),
        ],
        out_specs=[pl.BlockSpec((1, N), lambda i: (0, 0)),
                   pl.BlockSpec((8, 512), lambda i: (0, 0))],
        out_shape=[jax.ShapeDtypeStruct((1, N), jnp.float32),
                   jax.ShapeDtypeStruct((8, 512), jnp.float32)],
        scratch_shapes=[pltpu.VMEM((1, N), jnp.float32)],
    )(c, cos, te8, t2048, t4096)


def _final_kernel(w_ref, cv_ref, tm_ref, o_ref, m_scr):
    i = pl.program_id(0)
    k = pl.program_id(1)

    @pl.when(k == 0)
    def _():
        o_ref[...] = jnp.zeros_like(o_ref)

    @pl.when(i == 0)
    def _():
        base = cv_ref[...]
        base = jnp.where(k & 1 == 1, _rroll(base, 512), base)
        base = jnp.where(k & 2 == 2, _rroll(base, 1024), base)
        m = base
        rows = 1
        while rows < 512:
            m = jnp.concatenate([m, _rroll(m, rows)], axis=0)
            rows *= 2
        m_scr[pl.ds(k * 512, 512), :] = m

    a = w_ref[...] * tm_ref[...]
    o_ref[...] += jnp.dot(a, m_scr[pl.ds(k * 512, 512), :],
                          preferred_element_type=jnp.float32,
                          precision=jax.lax.Precision.DEFAULT)


def _reconstruct(w, cvec, tm):
    bi = 1024
    bk = 512
    return pl.pallas_call(
        _final_kernel,
        grid=(NW // bi, N // bk),
        in_specs=[
            pl.BlockSpec((bi, bk), lambda i, k: (i, k)),
            pl.BlockSpec((1, N), lambda i, k: (0, 0)),
            pl.BlockSpec((bi, 1), lambda i, k: (i, 0)),
        ],
        out_specs=pl.BlockSpec((bi, N), lambda i, k: (i, 0)),
        out_shape=jax.ShapeDtypeStruct((NW, N), jnp.float32),
        scratch_shapes=[pltpu.VMEM((N, N), jnp.float32)],
    )(w, cvec, tm)


def kernel(weight):
    w = weight.reshape(NW, N).astype(jnp.float32)
    cos = jnp.asarray(_COS_NP)
    c_gram, te_col = _gram(w)
    cvec, tm8 = _small_stages(c_gram, cos, te_col.reshape(8, 512),
                              jnp.asarray(_T2048_NP), jnp.asarray(_T4096_NP))
    rec = _reconstruct(w, cvec, tm8.reshape(NW, 1))
    return rec.reshape(weight.shape)


# speedup vs baseline: 1.7399x; 1.0597x over previous
import numpy as np
import jax
import jax.numpy as jnp
from jax.experimental import pallas as pl
from jax.experimental.pallas import tpu as pltpu

N = 2048
NW = 4096
_HI = jax.lax.Precision.HIGHEST

_k = np.arange(N)
_COS_NP = np.cos((2.0 * np.pi / N) * (np.outer(_k, _k) % N)).astype(np.float32)
_T2048_NP = np.asarray([max(1, int(c * (1.0 - 0.3))) for c in range(N + 1)],
                       dtype=np.float32).reshape(1, N + 1)
_T4096_NP = np.asarray([max(1, int(c * (1.0 - 0.3))) for c in range(NW + 1)],
                       dtype=np.float32).reshape(1, NW + 1)


def _gram_fold_kernel(a_ref, r_ref, te_ref, c_scr, s_scr):
    i = pl.program_id(0)
    dims = (((0,), (0,)), ((), ()))
    dflt = jax.lax.Precision.DEFAULT

    @pl.when(i == 0)
    def _():
        c_scr[...] = jnp.zeros_like(c_scr)
        s_scr[...] = jnp.zeros_like(s_scr)
        r_ref[...] = jnp.zeros_like(r_ref)

    @pl.when(i <= 7)
    def _():
        a = a_ref[...]
        hi = a.astype(jnp.bfloat16).astype(jnp.float32)
        lo = a - hi
        c_scr[...] += jax.lax.dot_general(
            hi, hi, dims, preferred_element_type=jnp.float32, precision=dflt)
        s_scr[...] += jax.lax.dot_general(
            hi, lo, dims, preferred_element_type=jnp.float32, precision=dflt)
        te_ref[pl.ds(i, 1), :] = jax.lax.dot_general(
            jnp.ones((1, N), jnp.float32), a * a, (((1,), (1,)), ((), ())),
            preferred_element_type=jnp.float32, precision=_HI)

    @pl.when(i >= 8)
    def _():
        j = i - 8
        row = pl.ds(j * 256, 256)
        st = [jnp.transpose(s_scr[pl.ds(c0, 512), row])
              for c0 in range(0, N, 512)]
        x = c_scr[row, :] + s_scr[row, :] + jnp.concatenate(st, axis=1)
        r_ref[...] += _fold_block(x, j, 256)


def _gram_fold(w):
    bk = 512
    return pl.pallas_call(
        _gram_fold_kernel,
        grid=(16,),
        in_specs=[pl.BlockSpec((bk, N), lambda i: (jnp.minimum(i, 7), 0))],
        out_specs=[pl.BlockSpec((1, N), lambda i: (0, 0)),
                   pl.BlockSpec((8, bk), lambda i: (0, 0))],
        out_shape=[jax.ShapeDtypeStruct((1, N), jnp.float32),
                   jax.ShapeDtypeStruct((8, bk), jnp.float32)],
        scratch_shapes=[pltpu.VMEM((N, N), jnp.float32),
                        pltpu.VMEM((N, N), jnp.float32)],
    )(w)


def _lroll(x, s):
    return jnp.concatenate([x[:, s:], x[:, :s]], axis=1)


def _rroll(x, s):
    return jnp.concatenate([x[:, N - s:], x[:, :N - s]], axis=1)


def _fold_block(x, blk, rows):
    while rows > 1:
        half = rows // 2
        top = jax.lax.slice(x, (0, 0), (half, N))
        bot = jax.lax.slice(x, (half, 0), (rows, N))
        x = top + _lroll(bot, half)
        rows = half
    for t in (8, 9, 10):
        bit = (blk >> (t - 8)) & 1
        x = jnp.where(bit == 1, _lroll(x, 1 << t), x)
    return x


def _ordered_key(v):
    i = jax.lax.bitcast_convert_type(v, jnp.int32)
    return jnp.where(i >= 0, i, i ^ jnp.int32(0x7FFFFFFF))


def _prefix_excl(eq, s0, s1):
    p = eq
    s = 1
    while s < s1:
        shifted = jnp.concatenate(
            [jnp.zeros((s0, s), jnp.float32), jax.lax.slice(p, (0, 0), (s0, s1 - s))],
            axis=1)
        p = p + shifted
        s *= 2
    rs = jnp.sum(eq, axis=1, keepdims=True)
    rp = rs
    s = 1
    while s < s0:
        shifted = jnp.concatenate(
            [jnp.zeros((s, 1), jnp.float32), jax.lax.slice(rp, (0, 0), (s0 - s, 1))],
            axis=0)
        rp = rp + shifted
        s *= 2
    return (p - eq) + (rp - rs)


def _topk_mask(v, table, L, s0, s1):
    mx = jnp.max(v)
    thresh = (v > jnp.float32(0.01) * mx)
    cnt = jnp.sum(thresh.astype(jnp.int32))
    ti = jax.lax.broadcasted_iota(jnp.int32, (1, L + 1), 1)
    target = jnp.sum(jnp.where(ti == cnt, table, 0.0)).astype(jnp.int32)
    key = _ordered_key(v)

    def body(_, lohi):
        lo, hi = lohi
        mid = lo + (hi - lo) // 2
        cgt = jnp.sum((key >= mid).astype(jnp.int32))
        pred = cgt >= target
        return (jnp.where(pred, mid, lo), jnp.where(pred, hi, mid))

    lo0 = jnp.min(key)
    hi0 = jnp.max(key) + 1
    tau, _ = jax.lax.fori_loop(0, 32, body, (lo0, hi0))
    gt = key > tau
    eq = (key == tau).astype(jnp.float32)
    fill = (target - jnp.sum(gt.astype(jnp.int32))).astype(jnp.float32)
    prefix = _prefix_excl(eq, s0, s1)
    mask_top = (gt | ((eq > 0) & (prefix < fill))).astype(jnp.float32)
    return jnp.where(target < cnt, mask_top, thresh.astype(jnp.float32))


def _matvec_cos(vec, cos_ref, blk=512):
    parts = []
    for j0 in range(0, N, blk):
        acc = jnp.zeros((1, blk), jnp.float32)
        for k0 in range(0, N, blk):
            vs = jax.lax.slice(vec, (0, k0), (1, k0 + blk))
            acc += jnp.dot(vs, cos_ref[k0:k0 + blk, j0:j0 + blk],
                           preferred_element_type=jnp.float32, precision=_HI)
        parts.append(acc)
    return jnp.concatenate(parts, axis=1)


def _small_kernel(r_ref, cos_ref, te8_ref, t2048_ref, t4096_ref,
                  cv_ref, tm_ref):
    g = _matvec_cos(r_ref[...], cos_ref)
    fm = _topk_mask(g, t2048_ref[...], N, 1, N)
    cv_ref[...] = _matvec_cos(fm, cos_ref) * jnp.float32(1.0 / N)
    tm_ref[...] = _topk_mask(te8_ref[...], t4096_ref[...], NW, 8, 512)


def _small_stages(r, cos, te8, t2048, t4096):
    return pl.pallas_call(
        _small_kernel,
        in_specs=[
            pl.BlockSpec((1, N), lambda: (0, 0)),
            pl.BlockSpec((N, N), lambda: (0, 0)),
            pl.BlockSpec((8, 512), lambda: (0, 0)),
            pl.BlockSpec((1, N + 1), lambda: (0, 0)),
            pl.BlockSpec((1, NW + 1), lambda: (0, 0)),
        ],
        out_specs=[pl.BlockSpec((1, N), lambda: (0, 0)),
                   pl.BlockSpec((8, 512), lambda: (0, 0))],
        out_shape=[jax.ShapeDtypeStruct((1, N), jnp.float32),
                   jax.ShapeDtypeStruct((8, 512), jnp.float32)],
    )(r, cos, te8, t2048, t4096)


def _final_kernel(w_ref, cv_ref, tm_ref, o_ref, m_scr):
    i = pl.program_id(0)
    k = pl.program_id(1)

    @pl.when(k == 0)
    def _():
        o_ref[...] = jnp.zeros_like(o_ref)

    @pl.when(i == 0)
    def _():
        base = cv_ref[...]
        base = jnp.where(k & 1 == 1, _rroll(base, 512), base)
        base = jnp.where(k & 2 == 2, _rroll(base, 1024), base)
        m = base
        rows = 1
        while rows < 512:
            m = jnp.concatenate([m, _rroll(m, rows)], axis=0)
            rows *= 2
        m_scr[pl.ds(k * 512, 512), :] = m

    a = w_ref[...] * tm_ref[...]
    o_ref[...] += jnp.dot(a, m_scr[pl.ds(k * 512, 512), :],
                          preferred_element_type=jnp.float32,
                          precision=jax.lax.Precision.DEFAULT)


def _reconstruct(w, cvec, tm):
    bi = 1024
    bk = 512
    return pl.pallas_call(
        _final_kernel,
        grid=(NW // bi, N // bk),
        in_specs=[
            pl.BlockSpec((bi, bk), lambda i, k: (i, k)),
            pl.BlockSpec((1, N), lambda i, k: (0, 0)),
            pl.BlockSpec((bi, 1), lambda i, k: (i, 0)),
        ],
        out_specs=pl.BlockSpec((bi, N), lambda i, k: (i, 0)),
        out_shape=jax.ShapeDtypeStruct((NW, N), jnp.float32),
        scratch_shapes=[pltpu.VMEM((N, N), jnp.float32)],
    )(w, cvec, tm)


def kernel(weight):
    w = weight.reshape(NW, N).astype(jnp.float32)
    cos = jnp.asarray(_COS_NP)
    r, te8 = _gram_fold(w)
    cvec, tm8 = _small_stages(r, cos, te8,
                              jnp.asarray(_T2048_NP), jnp.asarray(_T4096_NP))
    rec = _reconstruct(w, cvec, tm8.reshape(NW, 1))
    return rec.reshape(weight.shape)
